# Initial kernel scaffold; baseline (speedup 1.0000x reference)
import jax
import jax.numpy as jnp
from jax.experimental import pallas as pl


def kernel(x, X, Y):
    raise NotImplementedError("write your pallas kernel here")


# speedup vs baseline: 4.5858x; 4.5858x over previous
import functools

import jax
import jax.numpy as jnp
from jax.experimental import pallas as pl
from jax.experimental.pallas import tpu as pltpu

BATCH = 1024
DIM_IN = 256
DIM_OUT = 256
NUM_X = 64


def _interp_matmul_kernel(params_ref, x_ref, y_ref, out_ref, u_ref, uc_ref):
    k = pl.program_id(0)

    @pl.when(k == 0)
    def _init():
        xmin = params_ref[0, 0]
        inv_h = params_ref[0, 1]
        u = (x_ref[...] - xmin) * inv_h
        u_ref[...] = u
        uc_ref[...] = jnp.clip(u, 0.0, float(NUM_X - 1))

    kf = k.astype(jnp.float32)
    coeff = jnp.maximum(1.0 - jnp.abs(uc_ref[...] - kf), 0.0)

    coeff = _apply_edges(coeff, u_ref[...], k)

    acc = jax.lax.dot_general(
        coeff.astype(jnp.bfloat16),
        y_ref[0],
        (((1,), (0,)), ((), ())),
        preferred_element_type=jnp.float32,
    )

    @pl.when(k == 0)
    def _first():
        out_ref[...] = acc

    @pl.when(k > 0)
    def _rest():
        out_ref[...] += acc


def _apply_edges(coeff, u, k):
    def left(c):
        e0 = jnp.minimum(u, 0.0)
        sign = jnp.where(k == 0, -1.0, 1.0)
        return c + sign * e0

    def right(c):
        e1 = jnp.maximum(u - float(NUM_X - 1), 0.0)
        sign = jnp.where(k == NUM_X - 1, 1.0, -1.0)
        return c + sign * e1

    is_left = jnp.logical_or(k == 0, k == 1)
    is_right = jnp.logical_or(k == NUM_X - 2, k == NUM_X - 1)
    coeff = jax.lax.cond(is_left, left, lambda c: c, coeff)
    coeff = jax.lax.cond(is_right, right, lambda c: c, coeff)
    return coeff


@jax.jit
def kernel(x, X, Y):
    xmin = X[0]
    inv_h = (NUM_X - 1) / (X[NUM_X - 1] - X[0])
    params = jnp.stack([xmin, inv_h]).reshape(1, 2)
    yt = jnp.transpose(Y, (2, 0, 1)).astype(jnp.bfloat16)

    out = pl.pallas_call(
        _interp_matmul_kernel,
        grid=(NUM_X,),
        in_specs=[
            pl.BlockSpec(memory_space=pltpu.SMEM),
            pl.BlockSpec((BATCH, DIM_IN), lambda k: (0, 0)),
            pl.BlockSpec((1, DIM_IN, DIM_OUT), lambda k: (k, 0, 0)),
        ],
        out_specs=pl.BlockSpec((BATCH, DIM_OUT), lambda k: (0, 0)),
        out_shape=jax.ShapeDtypeStruct((BATCH, DIM_OUT), jnp.float32),
        scratch_shapes=[
            pltpu.VMEM((BATCH, DIM_IN), jnp.float32),
            pltpu.VMEM((BATCH, DIM_IN), jnp.float32),
        ],
    )(params, x, yt)
    return out


# speedup vs baseline: 13.8559x; 3.0214x over previous
import jax
import jax.numpy as jnp
from jax.experimental import pallas as pl
from jax.experimental.pallas import tpu as pltpu

BATCH = 1024
DIM_IN = 256
DIM_OUT = 256
NUM_X = 64
KB = 8
NSTEPS = NUM_X // KB


def _interp_matmul_kernel(params_ref, x_ref, y_ref, out_ref):
    s = pl.program_id(0)
    xmin = params_ref[0, 0]
    inv_h = params_ref[0, 1]
    u = (x_ref[...] - xmin) * inv_h
    uc = jnp.clip(u, 0.0, float(NUM_X - 1))
    base = (s * KB).astype(jnp.float32)

    hats = []
    for j in range(KB):
        d = (uc - (base + float(j))).astype(jnp.bfloat16)
        hats.append(jnp.maximum(1.0 - jnp.abs(d), 0.0))
    coeff = jnp.concatenate(hats, axis=1)

    acc = jax.lax.dot_general(
        coeff,
        y_ref[...].reshape(KB * DIM_IN, DIM_OUT),
        (((1,), (0,)), ((), ())),
        preferred_element_type=jnp.float32,
    )

    @pl.when(s == 0)
    def _first():
        e0 = jnp.minimum(u, 0.0).astype(jnp.bfloat16)
        d0 = y_ref[1] - y_ref[0]
        corr = jax.lax.dot_general(
            e0, d0, (((1,), (0,)), ((), ())),
            preferred_element_type=jnp.float32,
        )
        out_ref[...] = acc + corr

    @pl.when(jnp.logical_and(s > 0, s < NSTEPS - 1))
    def _mid():
        out_ref[...] += acc

    @pl.when(s == NSTEPS - 1)
    def _last():
        e1 = jnp.maximum(u - float(NUM_X - 1), 0.0).astype(jnp.bfloat16)
        d1 = y_ref[KB - 1] - y_ref[KB - 2]
        corr = jax.lax.dot_general(
            e1, d1, (((1,), (0,)), ((), ())),
            preferred_element_type=jnp.float32,
        )
        out_ref[...] += acc + corr


@jax.jit
def kernel(x, X, Y):
    xmin = X[0]
    inv_h = (NUM_X - 1) / (X[NUM_X - 1] - X[0])
    params = jnp.stack([xmin, inv_h]).reshape(1, 2)
    yt = jnp.transpose(Y, (2, 0, 1)).astype(jnp.bfloat16)

    out = pl.pallas_call(
        _interp_matmul_kernel,
        grid=(NSTEPS,),
        in_specs=[
            pl.BlockSpec(memory_space=pltpu.SMEM),
            pl.BlockSpec((BATCH, DIM_IN), lambda s: (0, 0)),
            pl.BlockSpec((KB, DIM_IN, DIM_OUT), lambda s: (s, 0, 0)),
        ],
        out_specs=pl.BlockSpec((BATCH, DIM_OUT), lambda s: (0, 0)),
        out_shape=jax.ShapeDtypeStruct((BATCH, DIM_OUT), jnp.float32),
    )(params, x, yt)
    return out


# speedup vs baseline: 14.2497x; 1.0284x over previous
import jax
import jax.numpy as jnp
from jax.experimental import pallas as pl
from jax.experimental.pallas import tpu as pltpu

BATCH = 1024
DIM_IN = 256
DIM_OUT = 256
NUM_X = 64
KB = 16
NSTEPS = NUM_X // KB


def _interp_matmul_kernel(params_ref, x_ref, y_ref, out_ref):
    s = pl.program_id(0)
    xmin = params_ref[0, 0]
    inv_h = params_ref[0, 1]
    u = (x_ref[...] - xmin) * inv_h
    uc = jnp.clip(u, 0.0, float(NUM_X - 1))
    base = (s * KB).astype(jnp.float32)

    hats = []
    for j in range(KB):
        d = (uc - (base + float(j))).astype(jnp.bfloat16)
        hats.append(jnp.maximum(1.0 - jnp.abs(d), 0.0))
    coeff = jnp.concatenate(hats, axis=1)

    acc = jax.lax.dot_general(
        coeff,
        y_ref[...].reshape(KB * DIM_IN, DIM_OUT),
        (((1,), (0,)), ((), ())),
        preferred_element_type=jnp.float32,
    )

    @pl.when(s == 0)
    def _first():
        e0 = jnp.minimum(u, 0.0).astype(jnp.bfloat16)
        d0 = y_ref[1] - y_ref[0]
        corr = jax.lax.dot_general(
            e0, d0, (((1,), (0,)), ((), ())),
            preferred_element_type=jnp.float32,
        )
        out_ref[...] = acc + corr

    @pl.when(jnp.logical_and(s > 0, s < NSTEPS - 1))
    def _mid():
        out_ref[...] += acc

    @pl.when(s == NSTEPS - 1)
    def _last():
        e1 = jnp.maximum(u - float(NUM_X - 1), 0.0).astype(jnp.bfloat16)
        d1 = y_ref[KB - 1] - y_ref[KB - 2]
        corr = jax.lax.dot_general(
            e1, d1, (((1,), (0,)), ((), ())),
            preferred_element_type=jnp.float32,
        )
        out_ref[...] += acc + corr


@jax.jit
def kernel(x, X, Y):
    xmin = X[0]
    inv_h = (NUM_X - 1) / (X[NUM_X - 1] - X[0])
    params = jnp.stack([xmin, inv_h]).reshape(1, 2)
    yt = jnp.transpose(Y, (2, 0, 1)).astype(jnp.bfloat16)

    out = pl.pallas_call(
        _interp_matmul_kernel,
        grid=(NSTEPS,),
        in_specs=[
            pl.BlockSpec(memory_space=pltpu.SMEM),
            pl.BlockSpec((BATCH, DIM_IN), lambda s: (0, 0)),
            pl.BlockSpec((KB, DIM_IN, DIM_OUT), lambda s: (s, 0, 0)),
        ],
        out_specs=pl.BlockSpec((BATCH, DIM_OUT), lambda s: (0, 0)),
        out_shape=jax.ShapeDtypeStruct((BATCH, DIM_OUT), jnp.float32),
    )(params, x, yt)
    return out
